# Initial kernel scaffold; baseline (speedup 1.0000x reference)
import jax
import jax.numpy as jnp
from jax.experimental import pallas as pl


def kernel(x, table):
    raise NotImplementedError("write your pallas kernel here")


# speedup vs baseline: 1.0923x; 1.0923x over previous
import functools

import jax
import jax.numpy as jnp
from jax import lax
from jax.experimental import pallas as pl
from jax.experimental.pallas import tpu as pltpu
from jax.experimental.pallas import tpu_sc as plsc

_B = 16384
_L = 50
_EMB = 32
_TOTAL = _B * _L

_NC = 2
_NS = 16
_NW = _NC * _NS
_PER_W = _TOTAL // _NW

_GSZ = 128
_CHUNK = 1024
_GPC = _CHUNK // _GSZ
_NCHUNK = _PER_W // _CHUNK

_mesh = plsc.VectorSubcoreMesh(core_axis_name="c", subcore_axis_name="s")


@functools.partial(
    pl.kernel,
    out_type=jax.ShapeDtypeStruct((_TOTAL, _EMB), jnp.float32),
    mesh=_mesh,
    scratch_types=[
        pltpu.VMEM((_GPC, _GSZ), jnp.int32),
        pltpu.VMEM((_CHUNK, _EMB), jnp.float32),
        pltpu.SemaphoreType.DMA,
    ],
    compiler_params=pltpu.CompilerParams(use_tc_tiling_on_sc=False),
)
def _emb_gather(idx_hbm, table_hbm, out_hbm, idx_v, rows_v, sem):
    wid = lax.axis_index("s") * _NC + lax.axis_index("c")
    base = wid * _PER_W
    idx_row0 = wid * (_PER_W // _GSZ)

    def step(c, carry):
        pltpu.sync_copy(idx_hbm.at[pl.ds(idx_row0 + c * _GPC, _GPC)], idx_v)
        copies = [
            pltpu.async_copy(
                table_hbm.at[idx_v.at[j]],
                rows_v.at[pl.ds(j * _GSZ, _GSZ)],
                sem,
            )
            for j in range(_GPC)
        ]
        for cp in copies:
            cp.wait()
        pltpu.sync_copy(rows_v, out_hbm.at[pl.ds(base + c * _CHUNK, _CHUNK)])
        return carry

    lax.fori_loop(0, _NCHUNK, step, 0)


def kernel(x, table):
    idx2d = x.reshape(_TOTAL // _GSZ, _GSZ)
    out = _emb_gather(idx2d, table)
    return out.reshape(_B, _L, _EMB)
